# baseline (speedup 1.0000x reference)
from kernel_sc_v4 import sc_kernel


def kernel(x):
    return sc_kernel(x)


# speedup vs baseline: 3.7616x; 3.7616x over previous
import math
import functools

import numpy as np
import jax
import jax.numpy as jnp
from jax.experimental import pallas as pl

_GRID = 32
_MAX_FRAMES = 32
_EMBED_DIM = 384
_SPATIAL_DIM = _EMBED_DIM * 2 // 3
_TEMPORAL_DIM = _EMBED_DIM - _SPATIAL_DIM
_X_DIM = _SPATIAL_DIM // 2
_Y_DIM = _SPATIAL_DIM - _X_DIM


def _create_pe_np(max_len, d):
    pos = np.arange(max_len, dtype=np.float32)[:, None]
    pe = np.zeros((max_len, d), dtype=np.float32)
    num_even = (d + 1) // 2
    num_odd = d // 2
    div_even = np.exp(np.arange(num_even, dtype=np.float32) * 2.0 * (-math.log(10000.0) / d))
    pe[:, 0::2] = np.sin(pos * div_even)
    if num_odd > 0:
        div_odd = np.exp(np.arange(num_odd, dtype=np.float32) * 2.0 * (-math.log(10000.0) / d))
        pe[:, 1::2] = np.cos(pos * div_odd)
    return pe


@functools.lru_cache(maxsize=None)
def _pe_tables(T):
    pe_x_tab = _create_pe_np(_GRID, _X_DIM)
    pe_y_tab = _create_pe_np(_GRID, _Y_DIM)
    pe_t_tab = _create_pe_np(_MAX_FRAMES, _TEMPORAL_DIM)
    yy, xx = np.meshgrid(np.arange(_GRID), np.arange(_GRID), indexing="ij")
    pe_x = pe_x_tab[xx.flatten()]
    pe_y = pe_y_tab[yy.flatten()]
    pe_spatial = np.concatenate([pe_x, pe_y], axis=-1)
    pe_t = pe_t_tab[:T][:, None, :]
    return jnp.asarray(pe_spatial), jnp.asarray(pe_t)


_TBLK = 8


def _add_pe_body(x_ref, ps_ref, pt_ref, o_ref):
    for i in range(_TBLK):
        xv = x_ref[0, i]
        o_ref[0, i, :, :_SPATIAL_DIM] = xv[:, :_SPATIAL_DIM] + ps_ref[...]
        o_ref[0, i, :, _SPATIAL_DIM:] = xv[:, _SPATIAL_DIM:] + pt_ref[i]


def kernel(x):
    B, T, N, E = x.shape
    pe_spatial, pe_t = _pe_tables(T)
    return pl.pallas_call(
        _add_pe_body,
        grid=(B, T // _TBLK),
        in_specs=[
            pl.BlockSpec((1, _TBLK, N, E), lambda b, t: (b, t, 0, 0)),
            pl.BlockSpec((N, _SPATIAL_DIM), lambda b, t: (0, 0)),
            pl.BlockSpec((_TBLK, 1, _TEMPORAL_DIM), lambda b, t: (t, 0, 0)),
        ],
        out_specs=pl.BlockSpec((1, _TBLK, N, E), lambda b, t: (b, t, 0, 0)),
        out_shape=jax.ShapeDtypeStruct((B, T, N, E), x.dtype),
    )(x, pe_spatial, pe_t)
